# baseline (device time: 38408 ns/iter reference)
import jax
import jax.numpy as jnp
from jax import lax
from jax.experimental import pallas as pl
from jax.experimental.pallas import tpu as pltpu

N_DEV = 4
M = 256
Q = 64
D = 256

RSR = 2


def BCR(par):
    return 3 + par


def kernel(x, Win0, Wout0, Win1, Wout1, Win2, Wout2):
    def body(x_ref, win0_ref, wout0_ref, win1_ref, wout1_ref, win2_ref,
             wout2_ref, out_ref, part0_ref, rbuf_ref, ag_stage_ref,
             rs_stage_ref, winb_ref, woutb_ref,
             ag_ssems, rs_ssems, bc_ssems, recv_sems):
        my = lax.axis_index("i")

        def rows(q):
            return pl.ds(q * Q, Q)

        barrier_sem = pltpu.get_barrier_semaphore()
        for d in (1, 2, 3):
            pl.semaphore_signal(
                barrier_sem, inc=1,
                device_id=(lax.rem(my + d, N_DEV),),
                device_id_type=pl.DeviceIdType.MESH,
            )
        pl.semaphore_wait(barrier_sem, N_DEV - 1)

        def ag_send_q(par, q):
            for d in (2, 1, 3):
                pltpu.make_async_remote_copy(
                    src_ref=ag_stage_ref.at[par, rows(q), :],
                    dst_ref=rbuf_ref.at[par, N_DEV - d, rows(q), :],
                    send_sem=ag_ssems.at[par, d - 1, q],
                    recv_sem=recv_sems.at[par, N_DEV - d, q],
                    device_id=(lax.rem(my + d, N_DEV),),
                    device_id_type=pl.DeviceIdType.MESH,
                ).start()

        def bc_send_q(par, q, ds=(2, 1, 3)):
            for d in ds:
                pltpu.make_async_remote_copy(
                    src_ref=rs_stage_ref.at[par, 1, rows(q), :],
                    dst_ref=rbuf_ref.at[BCR(par), N_DEV - d, rows(q), :],
                    send_sem=bc_ssems.at[par, d - 1, q],
                    recv_sem=recv_sems.at[BCR(par), N_DEV - d, q],
                    device_id=(lax.rem(my + d, N_DEV),),
                    device_id_type=pl.DeviceIdType.MESH,
                ).start()

        def rs_send_q(par, r, q):
            pltpu.make_async_remote_copy(
                src_ref=rs_stage_ref.at[par, r - 1, rows(q), :],
                dst_ref=rbuf_ref.at[RSR, N_DEV - r, rows(q), :],
                send_sem=rs_ssems.at[par, r - 1, q],
                recv_sem=recv_sems.at[RSR, N_DEV - r, q],
                device_id=(lax.rem(my + r, N_DEV),),
                device_id_type=pl.DeviceIdType.MESH,
            ).start()

        def _send_wait(sem_ref, src):
            pltpu.make_async_remote_copy(
                src_ref=src, dst_ref=src, send_sem=sem_ref,
                recv_sem=recv_sems.at[0, 0, 0],
                device_id=(my,), device_id_type=pl.DeviceIdType.MESH,
            ).wait_send()

        def ag_send_wait(par):
            for d in (1, 2, 3):
                for q in (0, 1, 2, 3):
                    _send_wait(ag_ssems.at[par, d - 1, q],
                               ag_stage_ref.at[par, rows(q), :])

        def bc_send_wait(par, ds=(1, 2, 3)):
            for d in ds:
                for q in (0, 1, 2, 3):
                    _send_wait(bc_ssems.at[par, d - 1, q],
                               rs_stage_ref.at[par, 1, rows(q), :])

        def rs_send_wait(par):
            for r in (1, 3):
                for q in (0, 1, 2, 3):
                    _send_wait(rs_ssems.at[par, r - 1, q],
                               rs_stage_ref.at[par, r - 1, rows(q), :])

        def recv_wait(region, slot, q):
            ref = rbuf_ref.at[region, slot, rows(q), :]
            pltpu.make_async_remote_copy(
                src_ref=ref, dst_ref=ref,
                send_sem=ag_ssems.at[0, 0, 0],
                recv_sem=recv_sems.at[region, slot, q],
                device_id=(my,), device_id_type=pl.DeviceIdType.MESH,
            ).wait_recv()

        for q in (0, 1, 2, 3):
            ag_stage_ref[0, rows(q), :] = x_ref[rows(q), :].astype(jnp.bfloat16)
            ag_send_q(0, q)

        layer_w = ((win0_ref, wout0_ref), (win1_ref, wout1_ref),
                   (win2_ref, wout2_ref))
        for l, (win_ref, wout_ref) in enumerate(layer_w):
            winb_ref[l, :, :] = win_ref[...].astype(jnp.bfloat16)
            woutb_ref[l, :, :] = wout_ref[...].astype(jnp.bfloat16)

        for l in range(3):
            s = l % 2
            s2 = (l + 1) % 2

            if l >= 2:
                rs_send_wait(s)
                bc_send_wait(s)

            def compute_pv(xv_bf16, l=l):
                hid = jnp.maximum(
                    jnp.dot(xv_bf16, winb_ref[l, :, :],
                            preferred_element_type=jnp.float32),
                    0.0,
                )
                return jnp.dot(hid.astype(jnp.bfloat16), woutb_ref[l, :, :],
                               preferred_element_type=jnp.float32)

            def item(r, q, l=l, s=s, s2=s2, compute_pv=compute_pv):
                if r == 0:
                    if l == 0:
                        xv = ag_stage_ref[s, rows(q), :]
                    else:
                        recv_wait(BCR(s2), 2, q)
                        xv = (
                            ag_stage_ref[s, rows(q), :].astype(jnp.float32)
                            + rbuf_ref[BCR(s2), 2, rows(q), :].astype(
                                jnp.float32)
                        ).astype(jnp.bfloat16)
                    part0_ref[rows(q), :] = compute_pv(xv)
                    return
                recv_wait(s, r, q)
                if r == 2:
                    if l == 0:
                        xv = rbuf_ref[s, 2, rows(q), :]
                    else:
                        xv = (
                            rbuf_ref[s, 2, rows(q), :].astype(jnp.float32)
                            + rs_stage_ref[s2, 1, rows(q), :].astype(
                                jnp.float32)
                        ).astype(jnp.bfloat16)
                    rs_stage_ref[s, 1, rows(q), :] = (
                        compute_pv(xv).astype(jnp.bfloat16))
                    bc_send_q(s, q, ds=(2, 1, 3) if l < 2 else (2,))
                    return
                if l == 0:
                    xv = rbuf_ref[s, r, rows(q), :]
                else:
                    xv = (
                        rbuf_ref[s, r, rows(q), :].astype(jnp.float32)
                        + rbuf_ref[BCR(s2), 4 - r, rows(q), :].astype(
                            jnp.float32)
                    ).astype(jnp.bfloat16)
                rs_stage_ref[s, r - 1, rows(q), :] = (
                    compute_pv(xv).astype(jnp.bfloat16))
                rs_send_q(s, r, q)

            if l == 0:
                for q in (0, 1, 2, 3):
                    item(0, q)
                order = ((1, 0), (3, 0), (1, 1), (3, 1), (2, 0),
                         (1, 2), (3, 2), (2, 1), (1, 3), (3, 3),
                         (2, 2), (2, 3))
            else:
                order = ((1, 0), (3, 0), (1, 1), (3, 1), (2, 0), (0, 0),
                         (1, 2), (3, 2), (2, 1), (0, 1), (1, 3), (3, 3),
                         (2, 2), (0, 2), (2, 3), (0, 3))
            for r, q in order:
                if l > 0 and r in (1, 3):
                    recv_wait(BCR(s2), 4 - r, q)
                item(r, q)

            if l >= 1:
                ag_send_wait(s2)
            for q in (0, 1, 2, 3):
                recv_wait(RSR, 1, q)
                recv_wait(RSR, 3, q)
                sv = (
                    part0_ref[rows(q), :]
                    + rbuf_ref[RSR, 1, rows(q), :].astype(jnp.float32)
                    + rbuf_ref[RSR, 3, rows(q), :].astype(jnp.float32)
                )
                if l < 2:
                    ag_stage_ref[s2, rows(q), :] = sv.astype(jnp.bfloat16)
                    ag_send_q(s2, q)
                else:
                    recv_wait(BCR(0), 2, q)
                    out_ref[rows(q), :] = sv + rbuf_ref[
                        BCR(0), 2, rows(q), :].astype(jnp.float32)

        rs_send_wait(0)
        rs_send_wait(1)
        bc_send_wait(0, ds=(2,))
        bc_send_wait(1)
        ag_send_wait(0)

    return pl.pallas_call(
        body,
        out_shape=jax.ShapeDtypeStruct((M, D), jnp.float32),
        in_specs=[pl.BlockSpec(memory_space=pltpu.VMEM)] * 7,
        out_specs=pl.BlockSpec(memory_space=pltpu.VMEM),
        scratch_shapes=[
            pltpu.VMEM((M, D), jnp.float32),
            pltpu.VMEM((5, N_DEV, M, D), jnp.bfloat16),
            pltpu.VMEM((2, M, D), jnp.bfloat16),
            pltpu.VMEM((2, 3, M, D), jnp.bfloat16),
            pltpu.VMEM((3, D, 2 * D), jnp.bfloat16),
            pltpu.VMEM((3, 2 * D, D), jnp.bfloat16),
            pltpu.SemaphoreType.DMA((2, 3, 4)),
            pltpu.SemaphoreType.DMA((2, 3, 4)),
            pltpu.SemaphoreType.DMA((2, 3, 4)),
            pltpu.SemaphoreType.DMA((5, N_DEV, 4)),
        ],
        compiler_params=pltpu.CompilerParams(collective_id=0),
    )(x, Win0, Wout0, Win1, Wout1, Win2, Wout2)


# device time: 35516 ns/iter; 1.0814x vs baseline; 1.0814x over previous
import jax
import jax.numpy as jnp
from jax import lax
from jax.experimental import pallas as pl
from jax.experimental.pallas import tpu as pltpu

N_DEV = 4
M = 256
Q = 64
D = 256


def kernel(x, Win0, Wout0, Win1, Wout1, Win2, Wout2):
    def body(x_ref, win0_ref, wout0_ref, win1_ref, wout1_ref, win2_ref,
             wout2_ref, out_ref, part0_ref, rbuf_ref, ag_stage_ref,
             rs_stage_ref, winb_ref, woutb_ref,
             ag_ssems, rs_ssems, recv_sems):
        my = lax.axis_index("i")

        def rows(q):
            return pl.ds(q * Q, Q)

        barrier_sem = pltpu.get_barrier_semaphore()
        for d in (1, 2, 3):
            pl.semaphore_signal(
                barrier_sem, inc=1,
                device_id=(lax.rem(my + d, N_DEV),),
                device_id_type=pl.DeviceIdType.MESH,
            )
        pl.semaphore_wait(barrier_sem, N_DEV - 1)

        def ag_send_q(s, q):
            for d in (2, 1, 3):
                rdma = pltpu.make_async_remote_copy(
                    src_ref=ag_stage_ref.at[s, rows(q), :],
                    dst_ref=rbuf_ref.at[0, N_DEV - d, rows(q), :],
                    send_sem=ag_ssems.at[s, d - 1, q],
                    recv_sem=recv_sems.at[0, N_DEV - d, q],
                    device_id=(lax.rem(my + d, N_DEV),),
                    device_id_type=pl.DeviceIdType.MESH,
                )
                rdma.start()

        def ag_send_wait(s):
            for d in (1, 2, 3):
                for q in (0, 1, 2, 3):
                    ref = ag_stage_ref.at[s, rows(q), :]
                    pltpu.make_async_remote_copy(
                        src_ref=ref, dst_ref=ref,
                        send_sem=ag_ssems.at[s, d - 1, q],
                        recv_sem=recv_sems.at[0, 0, 0],
                        device_id=(my,),
                        device_id_type=pl.DeviceIdType.MESH,
                    ).wait_send()

        def rs_send_wait():
            for r in (1, 2, 3):
                for q in (0, 1, 2, 3):
                    ref = rs_stage_ref.at[r - 1, rows(q), :]
                    pltpu.make_async_remote_copy(
                        src_ref=ref, dst_ref=ref,
                        send_sem=rs_ssems.at[r - 1, q],
                        recv_sem=recv_sems.at[0, 0, 0],
                        device_id=(my,),
                        device_id_type=pl.DeviceIdType.MESH,
                    ).wait_send()

        def recv_wait(par, slot, q):
            ref = rbuf_ref.at[par, slot, rows(q), :]
            pltpu.make_async_remote_copy(
                src_ref=ref, dst_ref=ref,
                send_sem=ag_ssems.at[0, 0, 0],
                recv_sem=recv_sems.at[par, slot, q],
                device_id=(my,),
                device_id_type=pl.DeviceIdType.MESH,
            ).wait_recv()

        for q in (0, 1, 2, 3):
            ag_stage_ref[0, rows(q), :] = x_ref[rows(q), :].astype(jnp.bfloat16)
            ag_send_q(0, q)

        layers = ((win0_ref, wout0_ref), (win1_ref, wout1_ref),
                  (win2_ref, wout2_ref))
        for l, (win_ref, wout_ref) in enumerate(layers):
            winb_ref[l, :, :] = win_ref[...].astype(jnp.bfloat16)
            woutb_ref[l, :, :] = wout_ref[...].astype(jnp.bfloat16)

        for l in range(3):
            s = l % 2
            s2 = (l + 1) % 2

            if l >= 1:
                rs_send_wait()

            def block_rows(xv_bf16, r, q, l=l):
                hid = jnp.maximum(
                    jnp.dot(xv_bf16, winb_ref[l, :, :],
                            preferred_element_type=jnp.float32),
                    0.0,
                )
                pv = jnp.dot(hid.astype(jnp.bfloat16), woutb_ref[l, :, :],
                             preferred_element_type=jnp.float32)
                if r == 0:
                    part0_ref[rows(q), :] = pv
                else:
                    rs_stage_ref[r - 1, rows(q), :] = pv.astype(jnp.bfloat16)

            if l == 0:
                for q in (0, 1, 2, 3):
                    block_rows(ag_stage_ref[s, rows(q), :], 0, q)

            for r, q in ((1, 0), (3, 0), (1, 1), (3, 1), (2, 0),
                         (1, 2), (3, 2), (2, 1), (1, 3), (3, 3),
                         (2, 2), (2, 3)):
                recv_wait(0, r, q)
                block_rows(rbuf_ref[0, r, rows(q), :], r, q)
                rdma = pltpu.make_async_remote_copy(
                    src_ref=rs_stage_ref.at[r - 1, rows(q), :],
                    dst_ref=rbuf_ref.at[1, N_DEV - r, rows(q), :],
                    send_sem=rs_ssems.at[r - 1, q],
                    recv_sem=recv_sems.at[1, N_DEV - r, q],
                    device_id=(lax.rem(my + r, N_DEV),),
                    device_id_type=pl.DeviceIdType.MESH,
                )
                rdma.start()

            if l >= 1:
                ag_send_wait(s2)
            for q in (0, 1, 2, 3):
                recv_wait(1, 1, q)
                recv_wait(1, 3, q)
                psum = (
                    part0_ref[rows(q), :]
                    + rbuf_ref[1, 1, rows(q), :].astype(jnp.float32)
                    + rbuf_ref[1, 3, rows(q), :].astype(jnp.float32)
                )
                recv_wait(1, 2, q)
                reduced = psum + rbuf_ref[1, 2, rows(q), :].astype(jnp.float32)
                if l < 2:
                    ag_stage_ref[s2, rows(q), :] = reduced.astype(jnp.bfloat16)
                    ag_send_q(s2, q)
                    hid = jnp.maximum(
                        jnp.dot(ag_stage_ref[s2, rows(q), :],
                                winb_ref[l + 1, :, :],
                                preferred_element_type=jnp.float32),
                        0.0,
                    )
                    part0_ref[rows(q), :] = jnp.dot(
                        hid.astype(jnp.bfloat16), woutb_ref[l + 1, :, :],
                        preferred_element_type=jnp.float32)
                else:
                    out_ref[rows(q), :] = reduced

        rs_send_wait()
        ag_send_wait(0)

    return pl.pallas_call(
        body,
        out_shape=jax.ShapeDtypeStruct((M, D), jnp.float32),
        in_specs=[pl.BlockSpec(memory_space=pltpu.VMEM)] * 7,
        out_specs=pl.BlockSpec(memory_space=pltpu.VMEM),
        scratch_shapes=[
            pltpu.VMEM((M, D), jnp.float32),
            pltpu.VMEM((2, N_DEV, M, D), jnp.bfloat16),
            pltpu.VMEM((2, M, D), jnp.bfloat16),
            pltpu.VMEM((3, M, D), jnp.bfloat16),
            pltpu.VMEM((3, D, 2 * D), jnp.bfloat16),
            pltpu.VMEM((3, 2 * D, D), jnp.bfloat16),
            pltpu.SemaphoreType.DMA((2, 3, 4)),
            pltpu.SemaphoreType.DMA((3, 4)),
            pltpu.SemaphoreType.DMA((2, N_DEV, 4)),
        ],
        compiler_params=pltpu.CompilerParams(collective_id=0),
    )(x, Win0, Wout0, Win1, Wout1, Win2, Wout2)
